# Initial kernel scaffold; baseline (speedup 1.0000x reference)
import jax
import jax.numpy as jnp
from jax.experimental import pallas as pl


def kernel(a1, a2, ft):
    raise NotImplementedError("write your pallas kernel here")


# speedup vs baseline: 1.8492x; 1.8492x over previous
import functools
import jax
import jax.numpy as jnp
from jax.experimental import pallas as pl
from jax.experimental.pallas import tpu as pltpu
from jax.experimental.pallas import tpu_sc as plsc

_N = 10000
_D = 16
_F = 256
_NSC = 1600
_SC_NB = 4
_TC_BN = 200
_LANES = 16


def _tc_body(a1_ref, a2_ref, ft_ref, out_ref):
    a = a1_ref[...][None, :, :] + a2_ref[...]
    a = jnp.maximum(a, 0.01 * a)
    m = jnp.max(a, axis=0, keepdims=True)
    e = jnp.exp(a - m)
    s = jnp.sum(e, axis=0)
    w = jnp.sum(e * ft_ref[...], axis=0)
    out_ref[...] = w / s


def _tc_part(a1, a2, ft):
    n, f = a1.shape
    d = a2.shape[0]
    bn = _TC_BN
    off = _NSC // bn
    grid = ((n - _NSC) // bn,)
    return pl.pallas_call(
        _tc_body,
        grid=grid,
        in_specs=[
            pl.BlockSpec((bn, f), lambda i: (i + off, 0)),
            pl.BlockSpec((d, bn, f), lambda i: (0, i + off, 0)),
            pl.BlockSpec((d, bn, f), lambda i: (0, i + off, 0)),
        ],
        out_specs=pl.BlockSpec((bn, f), lambda i: (i, 0)),
        out_shape=jax.ShapeDtypeStruct((n - _NSC, f), a1.dtype),
    )(a1, a2, ft)


def _sc_compute_block(a1_v, a2_v, ft_v, out_v):
    nb = a1_v.shape[0]

    @pl.loop(0, nb)
    def _node(nn):
        @pl.loop(0, _F, step=_LANES)
        def _fvec(f0):
            sl = pl.ds(f0, _LANES)
            va1 = a1_v[nn, sl]
            xs = []
            m = None
            for d in range(_D):
                x = a2_v[d, nn, sl] + va1
                x = jnp.maximum(x, x * 0.01)
                xs.append(x)
                m = x if m is None else jnp.maximum(m, x)
            s = None
            acc = None
            for d in range(_D):
                e = jnp.exp(xs[d] - m)
                s = e if s is None else s + e
                w = e * ft_v[d, nn, sl]
                acc = w if acc is None else acc + w
            out_v[nn, sl] = acc / s


def _sc_part(a1, a2, ft):
    mesh = plsc.VectorSubcoreMesh(core_axis_name="c", subcore_axis_name="s")

    @functools.partial(
        pl.kernel,
        out_type=jax.ShapeDtypeStruct((_NSC, _F), jnp.float32),
        mesh=mesh,
    )
    def sc_kern(a1_hbm, a2_hbm, ft_hbm, out_hbm):
        pltpu.emit_pipeline(
            _sc_compute_block,
            grid=(_NSC // _SC_NB,),
            in_specs=[
                pl.BlockSpec((_SC_NB, _F), lambda i: (i, 0)),
                pl.BlockSpec((_D, _SC_NB, _F), lambda i: (0, i, 0)),
                pl.BlockSpec((_D, _SC_NB, _F), lambda i: (0, i, 0)),
            ],
            out_specs=[pl.BlockSpec((_SC_NB, _F), lambda i: (i, 0))],
            core_axis_name=("c", "s"),
            dimension_semantics=(pltpu.PARALLEL,),
        )(a1_hbm, a2_hbm, ft_hbm, out_hbm)

    return sc_kern(a1, a2, ft)


def kernel(a1, a2, ft):
    sc_out = _sc_part(a1, a2, ft)
    tc_out = _tc_part(a1, a2, ft)
    return jnp.concatenate([sc_out, tc_out], axis=0)


# speedup vs baseline: 1.8584x; 1.0049x over previous
import functools
import jax
import jax.numpy as jnp
from jax.experimental import pallas as pl
from jax.experimental.pallas import tpu as pltpu
from jax.experimental.pallas import tpu_sc as plsc

_N = 10000
_D = 16
_F = 256
_NSC = 2400
_SC_NB = 4
_TC_BN = 200
_LANES = 16


def _tc_body(a1_ref, a2_ref, ft_ref, out_ref):
    a = a1_ref[...][None, :, :] + a2_ref[...]
    a = jnp.maximum(a, 0.01 * a)
    m = jnp.max(a, axis=0, keepdims=True)
    e = jnp.exp(a - m)
    s = jnp.sum(e, axis=0)
    w = jnp.sum(e * ft_ref[...], axis=0)
    out_ref[...] = w / s


def _tc_part(a1, a2, ft):
    n, f = a1.shape
    d = a2.shape[0]
    bn = _TC_BN
    off = _NSC // bn
    grid = ((n - _NSC) // bn,)
    return pl.pallas_call(
        _tc_body,
        grid=grid,
        in_specs=[
            pl.BlockSpec((bn, f), lambda i: (i + off, 0)),
            pl.BlockSpec((d, bn, f), lambda i: (0, i + off, 0)),
            pl.BlockSpec((d, bn, f), lambda i: (0, i + off, 0)),
        ],
        out_specs=pl.BlockSpec((bn, f), lambda i: (i, 0)),
        out_shape=jax.ShapeDtypeStruct((n - _NSC, f), a1.dtype),
    )(a1, a2, ft)


def _sc_compute_block(a1_v, a2_v, ft_v, out_v):
    nb = a1_v.shape[0]

    @pl.loop(0, nb)
    def _node(nn):
        @pl.loop(0, _F, step=_LANES)
        def _fvec(f0):
            sl = pl.ds(f0, _LANES)
            va1 = a1_v[nn, sl]
            xs = []
            m = None
            for d in range(_D):
                x = a2_v[d, nn, sl] + va1
                x = jnp.maximum(x, x * 0.01)
                xs.append(x)
                m = x if m is None else jnp.maximum(m, x)
            s = None
            acc = None
            for d in range(_D):
                e = jnp.exp(xs[d] - m)
                s = e if s is None else s + e
                w = e * ft_v[d, nn, sl]
                acc = w if acc is None else acc + w
            out_v[nn, sl] = acc / s


def _sc_part(a1, a2, ft):
    mesh = plsc.VectorSubcoreMesh(core_axis_name="c", subcore_axis_name="s")

    @functools.partial(
        pl.kernel,
        out_type=jax.ShapeDtypeStruct((_NSC, _F), jnp.float32),
        mesh=mesh,
    )
    def sc_kern(a1_hbm, a2_hbm, ft_hbm, out_hbm):
        pltpu.emit_pipeline(
            _sc_compute_block,
            grid=(_NSC // _SC_NB,),
            in_specs=[
                pl.BlockSpec((_SC_NB, _F), lambda i: (i, 0)),
                pl.BlockSpec((_D, _SC_NB, _F), lambda i: (0, i, 0)),
                pl.BlockSpec((_D, _SC_NB, _F), lambda i: (0, i, 0)),
            ],
            out_specs=[pl.BlockSpec((_SC_NB, _F), lambda i: (i, 0))],
            core_axis_name=("c", "s"),
            dimension_semantics=(pltpu.PARALLEL,),
        )(a1_hbm, a2_hbm, ft_hbm, out_hbm)

    return sc_kern(a1, a2, ft)


def kernel(a1, a2, ft):
    sc_out = _sc_part(a1, a2, ft)
    tc_out = _tc_part(a1, a2, ft)
    return jnp.concatenate([sc_out, tc_out], axis=0)


# speedup vs baseline: 1.9677x; 1.0589x over previous
import functools
import jax
import jax.numpy as jnp
from jax.experimental import pallas as pl
from jax.experimental.pallas import tpu as pltpu
from jax.experimental.pallas import tpu_sc as plsc

_N = 10000
_D = 16
_F = 256
_NSC = 3000
_SC_NB = 4
_TC_BN = 200
_LANES = 16


def _tc_body(a1_ref, a2_ref, ft_ref, out_ref):
    a = a1_ref[...][None, :, :] + a2_ref[...]
    a = jnp.maximum(a, 0.01 * a)
    m = jnp.max(a, axis=0, keepdims=True)
    e = jnp.exp(a - m)
    s = jnp.sum(e, axis=0)
    w = jnp.sum(e * ft_ref[...], axis=0)
    out_ref[...] = w / s


def _tc_part(a1, a2, ft):
    n, f = a1.shape
    d = a2.shape[0]
    bn = _TC_BN
    off = _NSC // bn
    grid = ((n - _NSC) // bn,)
    return pl.pallas_call(
        _tc_body,
        grid=grid,
        in_specs=[
            pl.BlockSpec((bn, f), lambda i: (i + off, 0)),
            pl.BlockSpec((d, bn, f), lambda i: (0, i + off, 0)),
            pl.BlockSpec((d, bn, f), lambda i: (0, i + off, 0)),
        ],
        out_specs=pl.BlockSpec((bn, f), lambda i: (i + off, 0)),
        out_shape=jax.ShapeDtypeStruct((n, f), a1.dtype),
    )(a1, a2, ft)


def _sc_compute_block(a1_v, a2_v, ft_v, out_v):
    nb = a1_v.shape[0]

    @pl.loop(0, nb)
    def _node(nn):
        @pl.loop(0, _F, step=_LANES, unroll=2)
        def _fvec(f0):
            sl = pl.ds(f0, _LANES)
            va1 = a1_v[nn, sl]
            xs = []
            m = None
            for d in range(_D):
                x = a2_v[d, nn, sl] + va1
                x = jnp.maximum(x, x * 0.01)
                xs.append(x)
                m = x if m is None else jnp.maximum(m, x)
            s = None
            acc = None
            for d in range(_D):
                e = jnp.exp(xs[d] - m)
                s = e if s is None else s + e
                w = e * ft_v[d, nn, sl]
                acc = w if acc is None else acc + w
            out_v[nn, sl] = acc / s


def _sc_part(a1, a2, ft):
    mesh = plsc.VectorSubcoreMesh(core_axis_name="c", subcore_axis_name="s")

    @functools.partial(
        pl.kernel,
        out_type=jax.ShapeDtypeStruct((_NSC, _F), jnp.float32),
        mesh=mesh,
    )
    def sc_kern(a1_hbm, a2_hbm, ft_hbm, out_hbm):
        pltpu.emit_pipeline(
            _sc_compute_block,
            grid=(_NSC // _SC_NB,),
            in_specs=[
                pl.BlockSpec((_SC_NB, _F), lambda i: (i, 0)),
                pl.BlockSpec((_D, _SC_NB, _F), lambda i: (0, i, 0)),
                pl.BlockSpec((_D, _SC_NB, _F), lambda i: (0, i, 0)),
            ],
            out_specs=[pl.BlockSpec((_SC_NB, _F), lambda i: (i, 0))],
            core_axis_name=("c", "s"),
            dimension_semantics=(pltpu.PARALLEL,),
        )(a1_hbm, a2_hbm, ft_hbm, out_hbm)

    return sc_kern(a1, a2, ft)


def kernel(a1, a2, ft):
    sc_out = _sc_part(a1, a2, ft)
    tc_out = _tc_part(a1, a2, ft)
    return jax.lax.dynamic_update_slice(tc_out, sc_out, (0, 0))
